# baseline (speedup 1.0000x reference)
import jax
import jax.numpy as jnp
from jax.experimental import pallas as pl
from jax.experimental.pallas import tpu as pltpu

BM = 400


def _fused_kernel(x_ref, adj_ref, w1_ref, b1_ref, w2_ref, b2_ref,
                  out_ref, t_ref, u_ref):
    d = pl.program_id(0)
    i = pl.program_id(1)

    @pl.when((d == 0) & (i == 0))
    def _():
        t_ref[...] = jnp.dot(x_ref[...], w1_ref[...].T,
                             preferred_element_type=jnp.float32)

    @pl.when(d == 0)
    def _():
        h = jnp.dot(adj_ref[...], t_ref[...],
                    preferred_element_type=jnp.float32)
        h = jnp.maximum(h + b1_ref[...], 0.0)
        u_ref[pl.ds(i * BM, BM), :] = jnp.dot(
            h, w2_ref[...].T, preferred_element_type=jnp.float32)

    @pl.when(d == 1)
    def _():
        z = jnp.dot(adj_ref[...], u_ref[...],
                    preferred_element_type=jnp.float32)
        z = z + b2_ref[...]
        m = jnp.max(z, axis=1, keepdims=True)
        e = z - m
        lse = jnp.log(jnp.sum(jnp.exp(e), axis=1, keepdims=True))
        out_ref[...] = e - lse


@jax.jit
def kernel(x, adj, W1, b1, W2, b2):
    in_f = x.shape[1]
    hid = W1.shape[0]
    out_f = W2.shape[0]
    n = adj.shape[0]

    return pl.pallas_call(
        _fused_kernel,
        grid=(2, n // BM),
        out_shape=jax.ShapeDtypeStruct((n, out_f), jnp.float32),
        in_specs=[
            pl.BlockSpec((n, in_f), lambda d, i: (0, 0)),
            pl.BlockSpec((BM, n), lambda d, i: (i, 0)),
            pl.BlockSpec((hid, in_f), lambda d, i: (0, 0)),
            pl.BlockSpec((hid,), lambda d, i: (0,)),
            pl.BlockSpec((out_f, hid), lambda d, i: (0, 0)),
            pl.BlockSpec((out_f,), lambda d, i: (0,)),
        ],
        out_specs=pl.BlockSpec((BM, out_f), lambda d, i: (d * i, 0)),
        scratch_shapes=[
            pltpu.VMEM((n, hid), jnp.float32),
            pltpu.VMEM((n, out_f), jnp.float32),
        ],
    )(x, adj, W1, b1, W2, b2)
